# Initial kernel scaffold; baseline (speedup 1.0000x reference)
import jax
import jax.numpy as jnp
from jax.experimental import pallas as pl


def kernel(entity_emb, relation_emb, W0, b0, W1, b1, Wp1, bp1, Wp2, bp2, edge_index, head_ids, relation_ids, tail_ids):
    raise NotImplementedError("write your pallas kernel here")


# speedup vs baseline: 4.1108x; 4.1108x over previous
import functools

import jax
import jax.numpy as jnp
from jax import lax
from jax.experimental import pallas as pl
from jax.experimental.pallas import tpu as pltpu
from jax.experimental.pallas import tpu_sc as plsc

_NC = 2
_NS = 16
_NW = _NC * _NS

_K = 80


def _sc_scatter(x, row, col, np_, with_counts):
    n, d = x.shape
    e = row.shape[0]
    epw = e // _NW
    chunks = epw // _K
    rpt = np_ // _NS
    zchunks = rpt // _K

    mesh = plsc.VectorSubcoreMesh(core_axis_name="c", subcore_axis_name="s")
    out_type = [jax.ShapeDtypeStruct((_NC, np_, d), jnp.float32)]
    if with_counts:
        out_type += [jax.ShapeDtypeStruct((_NC, np_, d), jnp.float32)]

    scratch = [
        pltpu.VMEM((_K,), jnp.int32),
        pltpu.VMEM((_K,), jnp.int32),
        pltpu.VMEM((_K,), jnp.int32),
        pltpu.VMEM((_K, d), jnp.float32),
        pltpu.VMEM_SHARED((np_, d), jnp.float32),
        pltpu.SemaphoreType.DMA,
    ]

    def impl(x_hbm, row_hbm, col_hbm, sums, cnts,
             idx_row, idx_col, zidx, rows_v, ssum, sem):
        cid = lax.axis_index("c")
        sid = lax.axis_index("s")
        wid = cid * _NS + sid
        sbase = sid * rpt
        ebase = wid * epw

        def fill_rows(val):
            v16 = jnp.full((16,), val, jnp.float32)

            def fstep(i, carry):
                for q in range(d // 16):
                    rows_v[i, pl.ds(q * 16, 16)] = v16
                return carry

            lax.fori_loop(0, _K, fstep, 0)

        def set_zidx(j):
            base = sbase + j * _K
            lanes = lax.iota(jnp.int32, 16)
            for q in range(_K // 16):
                zidx[pl.ds(q * 16, 16)] = lanes + (base + q * 16)

        def zero_acc():
            fill_rows(0.0)

            def zstep(j, carry):
                set_zidx(j)
                pltpu.sync_copy(rows_v, ssum.at[zidx])
                return carry

            lax.fori_loop(0, zchunks, zstep, 0)

        def write_out(dst):
            def wstep(j, carry):
                set_zidx(j)
                pltpu.async_copy(ssum.at[zidx], rows_v, sem).wait()
                pltpu.sync_copy(rows_v, dst.at[cid, pl.ds(sbase + j * _K, _K)])
                return carry

            lax.fori_loop(0, zchunks, wstep, 0)

        zero_acc()
        plsc.subcore_barrier()

        def step(i, carry):
            off = ebase + i * _K
            pltpu.sync_copy(col_hbm.at[pl.ds(off, _K)], idx_col)
            pltpu.sync_copy(row_hbm.at[pl.ds(off, _K)], idx_row)
            pltpu.async_copy(x_hbm.at[idx_col], rows_v, sem).wait()
            pltpu.sync_copy(rows_v, ssum.at[idx_row], add=True)
            return carry

        lax.fori_loop(0, chunks, step, 0)
        plsc.subcore_barrier()
        write_out(sums)

        if with_counts:
            zero_acc()
            fill_rows(1.0)
            plsc.subcore_barrier()

            def cstep(i, carry):
                off = ebase + i * _K
                pltpu.sync_copy(row_hbm.at[pl.ds(off, _K)], idx_row)
                pltpu.sync_copy(rows_v, ssum.at[idx_row], add=True)
                return carry

            lax.fori_loop(0, chunks, cstep, 0)
            plsc.subcore_barrier()
            write_out(cnts)

    if with_counts:
        def body(x_hbm, row_hbm, col_hbm, sums, cnts,
                 idx_row, idx_col, zidx, rows_v, ssum, sem):
            impl(x_hbm, row_hbm, col_hbm, sums, cnts,
                 idx_row, idx_col, zidx, rows_v, ssum, sem)
    else:
        def body(x_hbm, row_hbm, col_hbm, sums,
                 idx_row, idx_col, zidx, rows_v, ssum, sem):
            impl(x_hbm, row_hbm, col_hbm, sums, None,
                 idx_row, idx_col, zidx, rows_v, ssum, sem)

    f = pl.kernel(body, out_type=tuple(out_type), mesh=mesh,
                  scratch_types=scratch)
    res = f(x, row, col)
    if with_counts:
        sums, cnts = res
        return sums[0], sums[1], cnts[0], cnts[1]
    (sums,) = res
    return sums[0], sums[1]


def _sc_gather(x, rel_emb, head_ids, rel_ids, tail_ids):
    n, d = x.shape
    b = head_ids.shape[0]
    per_w = b // _NW
    kc = 128
    chunks = per_w // kc

    mesh = plsc.VectorSubcoreMesh(core_axis_name="c", subcore_axis_name="s")
    out_type = tuple(jax.ShapeDtypeStruct((b, d), jnp.float32) for _ in range(3))
    scratch = [
        pltpu.VMEM((kc,), jnp.int32),
        pltpu.VMEM((kc, d), jnp.float32),
        pltpu.SemaphoreType.DMA,
    ]

    def body(x_hbm, rel_hbm, hid, rid, tid, ho, ro, to, idx_v, rows_v, sem):
        cid = lax.axis_index("c")
        sid = lax.axis_index("s")
        base = (cid * _NS + sid) * per_w

        def step(i, carry):
            off = base + i * kc
            for ids, src, dst in ((hid, x_hbm, ho), (rid, rel_hbm, ro),
                                  (tid, x_hbm, to)):
                pltpu.sync_copy(ids.at[pl.ds(off, kc)], idx_v)
                pltpu.async_copy(src.at[idx_v], rows_v, sem).wait()
                pltpu.sync_copy(rows_v, dst.at[pl.ds(off, kc)])
            return carry

        lax.fori_loop(0, chunks, step, 0)

    f = pl.kernel(body, out_type=out_type, mesh=mesh, scratch_types=scratch)
    return f(x, rel_emb, head_ids, rel_ids, tail_ids)


def _tc_layer(x, s0, s1, c0, c1, w, b2d):
    n, d = x.shape
    h = w.shape[1]
    bn = 1000

    def body(x_ref, s0_ref, s1_ref, c0_ref, c1_ref, w_ref, b_ref, o_ref):
        cnt = c0_ref[:, :1] + c1_ref[:, :1]
        mean = (s0_ref[...] + s1_ref[...]) / (cnt + 1e-8)
        acc = jnp.dot(x_ref[...], w_ref[:d, :], preferred_element_type=jnp.float32)
        acc = acc + jnp.dot(mean, w_ref[d:, :], preferred_element_type=jnp.float32)
        o_ref[...] = jnp.maximum(acc + b_ref[...], 0.0)

    return pl.pallas_call(
        body,
        grid=(n // bn,),
        in_specs=[
            pl.BlockSpec((bn, d), lambda i: (i, 0)),
            pl.BlockSpec((bn, d), lambda i: (i, 0)),
            pl.BlockSpec((bn, d), lambda i: (i, 0)),
            pl.BlockSpec((bn, d), lambda i: (i, 0)),
            pl.BlockSpec((bn, d), lambda i: (i, 0)),
            pl.BlockSpec((2 * d, h), lambda i: (0, 0)),
            pl.BlockSpec((1, h), lambda i: (0, 0)),
        ],
        out_specs=pl.BlockSpec((bn, h), lambda i: (i, 0)),
        out_shape=jax.ShapeDtypeStruct((n, h), jnp.float32),
    )(x, s0, s1, c0, c1, w, b2d)


def _tc_predict(hd, rl, tl, wp1, bp1_2d, wp2_row, bp2_2d):
    b, d = hd.shape
    h = wp1.shape[1]
    bb = 2048

    def body(h_ref, r_ref, t_ref, w_ref, b1_ref, w2_ref, b2_ref, o_ref):
        acc = jnp.dot(h_ref[...], w_ref[:d, :], preferred_element_type=jnp.float32)
        acc = acc + jnp.dot(r_ref[...], w_ref[d:2 * d, :], preferred_element_type=jnp.float32)
        acc = acc + jnp.dot(t_ref[...], w_ref[2 * d:, :], preferred_element_type=jnp.float32)
        hh = jnp.maximum(acc + b1_ref[...], 0.0)
        score = jnp.sum(hh * w2_ref[...], axis=1, keepdims=True) + b2_ref[...]
        o_ref[...] = jax.nn.sigmoid(score)

    return pl.pallas_call(
        body,
        grid=(b // bb,),
        in_specs=[
            pl.BlockSpec((bb, d), lambda i: (i, 0)),
            pl.BlockSpec((bb, d), lambda i: (i, 0)),
            pl.BlockSpec((bb, d), lambda i: (i, 0)),
            pl.BlockSpec((3 * d, h), lambda i: (0, 0)),
            pl.BlockSpec((1, h), lambda i: (0, 0)),
            pl.BlockSpec((1, h), lambda i: (0, 0)),
            pl.BlockSpec((1, 1), lambda i: (0, 0)),
        ],
        out_specs=pl.BlockSpec((bb, 1), lambda i: (i, 0)),
        out_shape=jax.ShapeDtypeStruct((b, 1), jnp.float32),
    )(hd, rl, tl, wp1, bp1_2d, wp2_row, bp2_2d)


def kernel(entity_emb, relation_emb, W0, b0, W1, b1, Wp1, bp1, Wp2, bp2,
           edge_index, head_ids, relation_ids, tail_ids):
    n, d = entity_emb.shape
    row = edge_index[0].astype(jnp.int32)
    col = edge_index[1].astype(jnp.int32)
    head_ids = head_ids.astype(jnp.int32)
    relation_ids = relation_ids.astype(jnp.int32)
    tail_ids = tail_ids.astype(jnp.int32)

    np_ = ((n + _K * _NS - 1) // (_K * _NS)) * (_K * _NS)

    s0, s1, c0, c1 = _sc_scatter(entity_emb, row, col, np_, True)
    x1 = _tc_layer(entity_emb, s0, s1, c0, c1, W0, b0.reshape(1, -1))
    s0b, s1b = _sc_scatter(x1, row, col, np_, False)
    x2 = _tc_layer(x1, s0b, s1b, c0, c1, W1, b1.reshape(1, -1))
    hd, rl, tl = _sc_gather(x2, relation_emb, head_ids, relation_ids, tail_ids)
    score = _tc_predict(hd, rl, tl, Wp1, bp1.reshape(1, -1),
                        Wp2.reshape(1, -1), bp2.reshape(1, 1))
    return score[:, 0]
